# Initial kernel scaffold; baseline (speedup 1.0000x reference)
import jax
import jax.numpy as jnp
from jax.experimental import pallas as pl


def kernel(x, embedding_weight):
    raise NotImplementedError("write your pallas kernel here")


# speedup vs baseline: 10.1085x; 10.1085x over previous
import functools

import jax
import jax.numpy as jnp
from jax import lax
from jax.experimental import pallas as pl
from jax.experimental.pallas import tpu as pltpu
from jax.experimental.pallas import tpu_sc as plsc

_K = 8192
_D = 32
_N = 8192
_RB = 1024
_CB = 1024
_BETA = 0.25

_NC = 2
_NS = 16
_NW = _NC * _NS
_BPW = _N // _NW
_IDXW = 128
_NIDX = _BPW // _IDXW
_KPS = _K // _NS


def _argmin_body(xt_ref, e_ref, idx_ref):
    xbt = xt_ref[...]
    x2 = jnp.sum(xbt * xbt, axis=0, keepdims=True)
    xbt_bf = xbt.astype(jnp.bfloat16).astype(jnp.float32)

    def step(j, carry):
        best_v, best_i = carry
        ec = e_ref[pl.ds(j * _CB, _CB), :]
        e2 = jnp.sum(ec * ec, axis=1, keepdims=True)
        mm = lax.dot_general(ec, xbt_bf, (((1,), (0,)), ((), ())),
                             preferred_element_type=jnp.float32)
        dist = (x2 + e2) - 2.0 * mm
        m = jnp.min(dist, axis=0, keepdims=True)
        ids = lax.broadcasted_iota(jnp.int32, (_CB, _RB), 0) + j * _CB
        cand = jnp.min(jnp.where(dist == m, ids, _K), axis=0, keepdims=True)
        take = m < best_v
        return jnp.where(take, m, best_v), jnp.where(take, cand, best_i)

    init = (jnp.full((1, _RB), jnp.inf, jnp.float32),
            jnp.zeros((1, _RB), jnp.int32))
    _, best_i = lax.fori_loop(0, _K // _CB, step, init)
    idx_ref[...] = best_i.reshape(1, 1, _RB)


def _tc_argmin(flat_xt, emb):
    return pl.pallas_call(
        _argmin_body,
        grid=(_N // _RB,),
        in_specs=[
            pl.BlockSpec((_D, _RB), lambda i: (0, i)),
            pl.BlockSpec((_K, _D), lambda i: (0, 0)),
        ],
        out_specs=pl.BlockSpec((1, 1, _RB), lambda i: (i, 0, 0)),
        out_shape=jax.ShapeDtypeStruct((_N // _RB, 1, _RB), jnp.int32),
    )(flat_xt, emb)


def _sc_body(x_hbm, e_hbm, idx_hbm, ones_hbm, zeros_hbm,
             q_hbm, cnt_hbm, lp_hbm,
             idx_v, rows_v, x_v, ones_v, acc_v, cnt_sh, sem):
    cid = lax.axis_index("c")
    sid = lax.axis_index("s")
    wid = sid * _NC + cid
    base = wid * _BPW

    pltpu.sync_copy(idx_hbm.at[pl.ds(wid * _NIDX, _NIDX)], idx_v)
    cp0 = pltpu.async_copy(e_hbm.at[idx_v.at[0]],
                           rows_v.at[pl.ds(0, _IDXW)], sem)
    cp1 = pltpu.async_copy(e_hbm.at[idx_v.at[1]],
                           rows_v.at[pl.ds(_IDXW, _IDXW)], sem)
    pltpu.sync_copy(x_hbm.at[pl.ds(base, _BPW)], x_v)
    cp0.wait()
    cp1.wait()

    def row_step(r, acc):
        for off in (0, 16):
            v = rows_v[r, pl.ds(off, 16)]
            xv = x_v[r, pl.ds(off, 16)]
            d = v - xv
            rows_v[r, pl.ds(off, 16)] = xv + d
            acc = acc + d * d
        return acc

    acc = lax.fori_loop(0, _BPW, row_step, jnp.zeros((16,), jnp.float32))
    acc_v[...] = acc
    pltpu.sync_copy(acc_v, lp_hbm.at[cid, sid])
    pltpu.sync_copy(rows_v, q_hbm.at[pl.ds(base, _BPW)])

    pltpu.sync_copy(ones_hbm, ones_v)
    pltpu.sync_copy(zeros_hbm.at[pl.ds(sid * _KPS, _KPS)],
                    cnt_sh.at[pl.ds(sid * _KPS, _KPS)])
    plsc.subcore_barrier()
    pltpu.sync_copy(ones_v, cnt_sh.at[idx_v.at[0]], add=True)
    pltpu.sync_copy(ones_v, cnt_sh.at[idx_v.at[1]], add=True)
    plsc.subcore_barrier()
    pltpu.sync_copy(cnt_sh.at[pl.ds(sid * _KPS, _KPS)],
                    cnt_hbm.at[cid, pl.ds(sid * _KPS, _KPS)])


def _sc_quantize(flat_x, emb, idx2d, ones_in, zeros_in):
    call = pl.kernel(
        _sc_body,
        out_type=[
            jax.ShapeDtypeStruct((_N, _D), jnp.float32),
            jax.ShapeDtypeStruct((_NC, _K, 16), jnp.float32),
            jax.ShapeDtypeStruct((_NC, _NS, 16), jnp.float32),
        ],
        mesh=plsc.VectorSubcoreMesh(core_axis_name="c", subcore_axis_name="s",
                                    num_cores=_NC, num_subcores=_NS),
        scratch_types=[
            pltpu.VMEM((_NIDX, _IDXW), jnp.int32),
            pltpu.VMEM((_BPW, _D), jnp.float32),
            pltpu.VMEM((_BPW, _D), jnp.float32),
            pltpu.VMEM((_IDXW, 16), jnp.float32),
            pltpu.VMEM((16,), jnp.float32),
            pltpu.VMEM_SHARED((_K, 16), jnp.float32),
            pltpu.SemaphoreType.DMA,
        ],
        compiler_params=pltpu.CompilerParams(use_tc_tiling_on_sc=False),
    )
    return call(flat_x, emb, idx2d, ones_in, zeros_in)


def _final_body(cnt_ref, lp_ref, loss_ref, perp_ref):
    c = cnt_ref[0] + cnt_ref[1]
    s = jnp.sum(c, axis=1, keepdims=True)
    mean = s * (1.0 / (16.0 * _N))
    ent = jnp.sum(mean * jnp.log(mean + 1e-10))
    perp_ref[0, 0] = jnp.exp(-ent)
    m = jnp.sum(lp_ref[...]) * (1.0 / (_N * _D))
    loss_ref[0, 0] = m * _BETA + m


def _tc_final(cnt, lp):
    return pl.pallas_call(
        _final_body,
        in_specs=[
            pl.BlockSpec(memory_space=pltpu.MemorySpace.VMEM),
            pl.BlockSpec(memory_space=pltpu.MemorySpace.VMEM),
        ],
        out_specs=[
            pl.BlockSpec(memory_space=pltpu.MemorySpace.SMEM),
            pl.BlockSpec(memory_space=pltpu.MemorySpace.SMEM),
        ],
        out_shape=[
            jax.ShapeDtypeStruct((1, 1), jnp.float32),
            jax.ShapeDtypeStruct((1, 1), jnp.float32),
        ],
    )(cnt, lp)


def kernel(x, embedding_weight):
    xp = jnp.transpose(x, (0, 2, 3, 1))
    flat_x = xp.reshape(_N, _D)
    flat_xt = jnp.transpose(flat_x, (1, 0))
    idx = _tc_argmin(flat_xt, embedding_weight)
    idx2d = idx.reshape(_N // _IDXW, _IDXW)
    ones_in = jnp.ones((_IDXW, 16), jnp.float32)
    zeros_in = jnp.zeros((_K, 16), jnp.float32)
    q, cnt, lp = _sc_quantize(flat_x, embedding_weight, idx2d,
                              ones_in, zeros_in)
    loss, perp = _tc_final(cnt, lp)
    qx = jnp.transpose(q.reshape(8, 32, 32, _D), (0, 3, 1, 2))
    return loss[0, 0], qx, perp[0, 0]


# speedup vs baseline: 10.1181x; 1.0009x over previous
import jax
import jax.numpy as jnp
from jax import lax
from jax.experimental import pallas as pl
from jax.experimental.pallas import tpu as pltpu
from jax.experimental.pallas import tpu_sc as plsc

_K = 8192
_D = 32
_N = 8192
_RB = 1024
_CB = 1024
_BETA = 0.25

_NC = 2
_NS = 16
_NW = _NC * _NS
_BPW = _N // _NW
_IDXW = 128
_NIDX = _BPW // _IDXW
_KPS = _K // _NS


def _argmin_body(xt_ref, e_ref, idx_ref):
    xbt = xt_ref[...]
    x2 = jnp.sum(xbt * xbt, axis=0, keepdims=True)
    xbt_bf = xbt.astype(jnp.bfloat16).astype(jnp.float32)

    def step(j, carry):
        best_v, best_i = carry
        ec = e_ref[pl.ds(j * _CB, _CB), :]
        e2 = jnp.sum(ec * ec, axis=1, keepdims=True)
        mm = lax.dot_general(ec, xbt_bf, (((1,), (0,)), ((), ())),
                             preferred_element_type=jnp.float32)
        dist = (x2 + e2) - 2.0 * mm
        m = jnp.min(dist, axis=0, keepdims=True)
        ids = lax.broadcasted_iota(jnp.int32, (_CB, _RB), 0) + j * _CB
        cand = jnp.min(jnp.where(dist == m, ids, _K), axis=0, keepdims=True)
        take = m < best_v
        return jnp.where(take, m, best_v), jnp.where(take, cand, best_i)

    init = (jnp.full((1, _RB), jnp.inf, jnp.float32),
            jnp.zeros((1, _RB), jnp.int32))
    _, best_i = lax.fori_loop(0, _K // _CB, step, init)
    idx_ref[...] = best_i.reshape(1, 1, _RB)


def _tc_argmin(flat_xt, emb):
    return pl.pallas_call(
        _argmin_body,
        grid=(_N // _RB,),
        in_specs=[
            pl.BlockSpec((_D, _RB), lambda i: (0, i)),
            pl.BlockSpec((_K, _D), lambda i: (0, 0)),
        ],
        out_specs=pl.BlockSpec((1, 1, _RB), lambda i: (i, 0, 0)),
        out_shape=jax.ShapeDtypeStruct((_N // _RB, 1, _RB), jnp.int32),
    )(flat_xt, emb)


def _sc_body(x_hbm, e_hbm, idx_hbm, ones_hbm, zeros_hbm,
             q_hbm, cnt_hbm, lp_hbm,
             idx_v, rows_v, x_v, ones_v, acc_v, cnt_sh, sem):
    cid = lax.axis_index("c")
    sid = lax.axis_index("s")
    wid = sid * _NC + cid
    base = wid * _BPW

    pltpu.sync_copy(idx_hbm.at[pl.ds(wid * _NIDX, _NIDX)], idx_v)
    cp0 = pltpu.async_copy(e_hbm.at[idx_v.at[0]],
                           rows_v.at[pl.ds(0, _IDXW)], sem)
    cp1 = pltpu.async_copy(e_hbm.at[idx_v.at[1]],
                           rows_v.at[pl.ds(_IDXW, _IDXW)], sem)
    pltpu.sync_copy(x_hbm.at[pl.ds(base, _BPW)], x_v)
    cp0.wait()
    cp1.wait()

    def row_step(r, acc):
        for off in (0, 16):
            v = rows_v[r, pl.ds(off, 16)]
            xv = x_v[r, pl.ds(off, 16)]
            d = v - xv
            rows_v[r, pl.ds(off, 16)] = xv + d
            acc = acc + d * d
        return acc

    acc = lax.fori_loop(0, _BPW, row_step, jnp.zeros((16,), jnp.float32))
    acc_v[...] = acc
    pltpu.sync_copy(acc_v, lp_hbm.at[cid, sid])
    pltpu.sync_copy(rows_v, q_hbm.at[pl.ds(base, _BPW)])

    pltpu.sync_copy(ones_hbm, ones_v)
    pltpu.sync_copy(zeros_hbm.at[pl.ds(sid * _KPS, _KPS)],
                    cnt_sh.at[pl.ds(sid * _KPS, _KPS)])
    plsc.subcore_barrier()
    pltpu.sync_copy(ones_v, cnt_sh.at[idx_v.at[0]], add=True)
    pltpu.sync_copy(ones_v, cnt_sh.at[idx_v.at[1]], add=True)
    plsc.subcore_barrier()
    pltpu.sync_copy(cnt_sh.at[pl.ds(sid * _KPS, _KPS)],
                    cnt_hbm.at[cid, pl.ds(sid * _KPS, _KPS)])


def _sc_quantize(flat_x, emb, idx2d, ones_in, zeros_in):
    call = pl.kernel(
        _sc_body,
        out_type=[
            jax.ShapeDtypeStruct((_N, _D), jnp.float32),
            jax.ShapeDtypeStruct((_NC, _K, 16), jnp.float32),
            jax.ShapeDtypeStruct((_NC, _NS, 16), jnp.float32),
        ],
        mesh=plsc.VectorSubcoreMesh(core_axis_name="c", subcore_axis_name="s",
                                    num_cores=_NC, num_subcores=_NS),
        scratch_types=[
            pltpu.VMEM((_NIDX, _IDXW), jnp.int32),
            pltpu.VMEM((_BPW, _D), jnp.float32),
            pltpu.VMEM((_BPW, _D), jnp.float32),
            pltpu.VMEM((_IDXW, 16), jnp.float32),
            pltpu.VMEM((16,), jnp.float32),
            pltpu.VMEM_SHARED((_K, 16), jnp.float32),
            pltpu.SemaphoreType.DMA,
        ],
        compiler_params=pltpu.CompilerParams(use_tc_tiling_on_sc=False),
    )
    return call(flat_x, emb, idx2d, ones_in, zeros_in)


def _final_body(cnt_ref, lp_ref, loss_ref, perp_ref):
    c = cnt_ref[0] + cnt_ref[1]
    s = jnp.sum(c, axis=1, keepdims=True)
    mean = s * (1.0 / (16.0 * _N))
    ent = jnp.sum(mean * jnp.log(mean + 1e-10))
    perp_ref[0, 0] = jnp.exp(-ent)
    m = jnp.sum(lp_ref[...]) * (1.0 / (_N * _D))
    loss_ref[0, 0] = m * _BETA + m


def _tc_final(cnt, lp):
    return pl.pallas_call(
        _final_body,
        in_specs=[
            pl.BlockSpec(memory_space=pltpu.MemorySpace.VMEM),
            pl.BlockSpec(memory_space=pltpu.MemorySpace.VMEM),
        ],
        out_specs=[
            pl.BlockSpec(memory_space=pltpu.MemorySpace.SMEM),
            pl.BlockSpec(memory_space=pltpu.MemorySpace.SMEM),
        ],
        out_shape=[
            jax.ShapeDtypeStruct((1, 1), jnp.float32),
            jax.ShapeDtypeStruct((1, 1), jnp.float32),
        ],
    )(cnt, lp)


def kernel(x, embedding_weight):
    xp = jnp.transpose(x, (0, 2, 3, 1))
    flat_x = xp.reshape(_N, _D)
    flat_xt = jnp.transpose(flat_x, (1, 0))
    idx = _tc_argmin(flat_xt, embedding_weight)
    idx2d = idx.reshape(_N // _IDXW, _IDXW)
    ones_in = jnp.ones((_IDXW, 16), jnp.float32)
    zeros_in = jnp.zeros((_K, 16), jnp.float32)
    q, cnt, lp = _sc_quantize(flat_x, embedding_weight, idx2d,
                              ones_in, zeros_in)
    loss, perp = _tc_final(cnt, lp)
    qx = jnp.transpose(q.reshape(8, 32, 32, _D), (0, 3, 1, 2))
    return loss[0, 0], qx, perp[0, 0]
